# baseline (speedup 1.0000x reference)
import functools

import jax
import jax.numpy as jnp
from jax import lax
from jax.experimental import pallas as pl
from jax.experimental.pallas import tpu as pltpu
from jax.experimental.pallas import tpu_sc as plsc

VOCAB = 100000
EMBED_DIM = 128
BATCH = 4096
HIST = 50
NUM_WORKERS = 32
B_PER_W = BATCH // NUM_WORKERS
CHUNK = 64
NCHUNK = HIST * 2
NBUF = 8
GROUPS = 12
TAIL = NCHUNK - GROUPS * NBUF


def _make_gather():
    mesh = plsc.VectorSubcoreMesh(core_axis_name="c", subcore_axis_name="s")

    scratch = [pltpu.VMEM((HIST, B_PER_W), jnp.int32)]
    scratch += [
        pltpu.VMEM((CHUNK, EMBED_DIM), jnp.float32) for _ in range(NBUF)
    ]
    scratch += [pltpu.SemaphoreType.DMA for _ in range(2 * NBUF + 1)]

    @functools.partial(
        pl.kernel,
        mesh=mesh,
        out_type=jax.ShapeDtypeStruct((HIST, BATCH, EMBED_DIM), jnp.float32),
        scratch_types=scratch,
    )
    def gather_kernel(idx_hbm, table_hbm, out_hbm, idx_v, *bufs_and_sems):
        bufs = bufs_and_sems[:NBUF]
        gsem = bufs_and_sems[NBUF:2 * NBUF]
        psem = bufs_and_sems[2 * NBUF:3 * NBUF]
        isem = bufs_and_sems[3 * NBUF]
        wid = lax.axis_index("s") * 2 + lax.axis_index("c")
        cbase = wid * B_PER_W
        pltpu.sync_copy(
            idx_hbm.at[pl.ds(0, 8), pl.ds(cbase, B_PER_W)],
            idx_v.at[pl.ds(0, 8)],
        )
        idx_rest = pltpu.async_copy(
            idx_hbm.at[pl.ds(8, HIST - 8), pl.ds(cbase, B_PER_W)],
            idx_v.at[pl.ds(8, HIST - 8)],
            isem,
        )

        def _slices(c):
            l = c // 2
            h = (c % 2) * CHUNK
            return l, h

        def fire_gather(c, s):
            l, h = _slices(c)
            pltpu.async_copy(
                table_hbm.at[idx_v.at[l, pl.ds(h, CHUNK)]], bufs[s], gsem[s]
            )

        def wait_gather(c, s):
            l, h = _slices(c)
            pltpu.make_async_copy(
                table_hbm.at[idx_v.at[l, pl.ds(h, CHUNK)]], bufs[s], gsem[s]
            ).wait()

        def fire_put(c, s):
            l, h = _slices(c)
            pltpu.async_copy(
                bufs[s], out_hbm.at[l, pl.ds(cbase + h, CHUNK)], psem[s]
            )

        def wait_put(c, s):
            l, h = _slices(c)
            pltpu.make_async_copy(
                bufs[s], out_hbm.at[l, pl.ds(cbase + h, CHUNK)], psem[s]
            ).wait()

        for s in range(NBUF):
            fire_gather(s, s)
        idx_rest.wait()

        def body(g, carry):
            for s in range(NBUF):
                l = g * NBUF + s
                wait_gather(l, s)
                fire_put(l, s)
            for s in range(NBUF):
                l = g * NBUF + s
                wait_put(l, s)
                fire_gather((g + 1) * NBUF + s, s)
            return carry

        lax.fori_loop(0, GROUPS - 1, body, 0)

        g = GROUPS - 1
        for s in range(NBUF):
            l = g * NBUF + s
            wait_gather(l, s)
            fire_put(l, s)
        for s in range(NBUF):
            wait_put(g * NBUF + s, s)
        for t in range(TAIL):
            fire_gather(GROUPS * NBUF + t, t)
        for t in range(TAIL):
            l = GROUPS * NBUF + t
            wait_gather(l, t)
            fire_put(l, t)
        for t in range(TAIL):
            wait_put(GROUPS * NBUF + t, t)

    return gather_kernel


_gather = _make_gather()


def kernel(vocab_id_list, table):
    idx_t = vocab_id_list.T
    out_t = _gather(idx_t, table)
    return out_t.transpose(1, 0, 2)


# speedup vs baseline: 1.0199x; 1.0199x over previous
import functools

import jax
import jax.numpy as jnp
from jax import lax
from jax.experimental import pallas as pl
from jax.experimental.pallas import tpu as pltpu
from jax.experimental.pallas import tpu_sc as plsc

VOCAB = 100000
EMBED_DIM = 128
BATCH = 4096
HIST = 50
NUM_WORKERS = 32
B_PER_W = BATCH // NUM_WORKERS
CHUNK = 64
NCHUNK = HIST * 2
NBUF = 10
GROUPS = NCHUNK // NBUF
TAIL = NCHUNK - GROUPS * NBUF


def _make_gather():
    mesh = plsc.VectorSubcoreMesh(core_axis_name="c", subcore_axis_name="s")

    scratch = [pltpu.VMEM((HIST, B_PER_W), jnp.int32)]
    scratch += [
        pltpu.VMEM((CHUNK, EMBED_DIM), jnp.float32) for _ in range(NBUF)
    ]
    scratch += [pltpu.SemaphoreType.DMA for _ in range(2 * NBUF + 1)]

    @functools.partial(
        pl.kernel,
        mesh=mesh,
        out_type=jax.ShapeDtypeStruct((HIST, BATCH, EMBED_DIM), jnp.float32),
        scratch_types=scratch,
    )
    def gather_kernel(idx_hbm, table_hbm, out_hbm, idx_v, *bufs_and_sems):
        bufs = bufs_and_sems[:NBUF]
        gsem = bufs_and_sems[NBUF:2 * NBUF]
        psem = bufs_and_sems[2 * NBUF:3 * NBUF]
        isem = bufs_and_sems[3 * NBUF]
        wid = lax.axis_index("s") * 2 + lax.axis_index("c")
        cbase = wid * B_PER_W
        pltpu.sync_copy(
            idx_hbm.at[pl.ds(0, 8), pl.ds(cbase, B_PER_W)],
            idx_v.at[pl.ds(0, 8)],
        )
        idx_rest = pltpu.async_copy(
            idx_hbm.at[pl.ds(8, HIST - 8), pl.ds(cbase, B_PER_W)],
            idx_v.at[pl.ds(8, HIST - 8)],
            isem,
        )

        def _slices(c):
            l = c // 2
            h = (c % 2) * CHUNK
            return l, h

        def fire_gather(c, s):
            l, h = _slices(c)
            pltpu.async_copy(
                table_hbm.at[idx_v.at[l, pl.ds(h, CHUNK)]], bufs[s], gsem[s]
            )

        def wait_gather(c, s):
            l, h = _slices(c)
            pltpu.make_async_copy(
                table_hbm.at[idx_v.at[l, pl.ds(h, CHUNK)]], bufs[s], gsem[s]
            ).wait()

        def fire_put(c, s):
            l, h = _slices(c)
            pltpu.async_copy(
                bufs[s], out_hbm.at[l, pl.ds(cbase + h, CHUNK)], psem[s]
            )

        def wait_put(c, s):
            l, h = _slices(c)
            pltpu.make_async_copy(
                bufs[s], out_hbm.at[l, pl.ds(cbase + h, CHUNK)], psem[s]
            ).wait()

        for s in range(NBUF):
            fire_gather(s, s)
        idx_rest.wait()

        def body(g, carry):
            for s in range(NBUF):
                l = g * NBUF + s
                wait_gather(l, s)
                fire_put(l, s)
            for s in range(NBUF):
                l = g * NBUF + s
                wait_put(l, s)
                fire_gather((g + 1) * NBUF + s, s)
            return carry

        lax.fori_loop(0, GROUPS - 1, body, 0)

        g = GROUPS - 1
        for s in range(NBUF):
            l = g * NBUF + s
            wait_gather(l, s)
            fire_put(l, s)
        for s in range(NBUF):
            wait_put(g * NBUF + s, s)
        for t in range(TAIL):
            fire_gather(GROUPS * NBUF + t, t)
        for t in range(TAIL):
            l = GROUPS * NBUF + t
            wait_gather(l, t)
            fire_put(l, t)
        for t in range(TAIL):
            wait_put(GROUPS * NBUF + t, t)

    return gather_kernel


_gather = _make_gather()


def kernel(vocab_id_list, table):
    idx_t = vocab_id_list.T
    out_t = _gather(idx_t, table)
    return out_t.transpose(1, 0, 2)
